# Initial kernel scaffold; baseline (speedup 1.0000x reference)
import jax
import jax.numpy as jnp
from jax.experimental import pallas as pl

_WIN_BASES = (560, 350, 120)
_WLEN = 20
_CBLK = 10
_NJ = _WLEN // _CBLK
_HBLK = 112


def _body(xref, oref):
    j = pl.program_id(3)
    s = jnp.sum(xref[0], axis=0) * (1.0 / _WLEN)

    @pl.when(j == 0)
    def _():
        oref[0, 0] = s

    @pl.when(j > 0)
    def _():
        oref[0, 0] += s


def kernel(x):
    B, C, H, W = x.shape
    bases10 = jnp.asarray([b // _CBLK for b in _WIN_BASES], dtype=jnp.int32)

    grid = (len(_WIN_BASES), B, H // _HBLK, _NJ)

    out = pl.pallas_call(
        _body,
        grid=grid,
        in_specs=[
            pl.BlockSpec(
                (1, _CBLK, _HBLK, W),
                lambda w, b, h, j: (b, bases10[w] + j, h, 0),
            )
        ],
        out_specs=pl.BlockSpec(
            (1, 1, _HBLK, W),
            lambda w, b, h, j: (b, w, h, 0),
        ),
        out_shape=jax.ShapeDtypeStruct((B, len(_WIN_BASES), H, W), x.dtype),
    )(x)
    return out


# speedup vs baseline: 4.2621x; 4.2621x over previous
import jax
import jax.numpy as jnp
from jax.experimental import pallas as pl

_WIN_BASES = (560, 350, 120)
_WLEN = 20
_CBLK = 10
_NJ = _WLEN // _CBLK
_HBLK = 112


def _body(xref, oref):
    j = pl.program_id(3)
    s = jnp.sum(xref[0], axis=0) * (1.0 / _WLEN)

    @pl.when(j == 0)
    def _():
        oref[0, 0] = s

    @pl.when(j > 0)
    def _():
        oref[0, 0] += s


def kernel(x):
    B, C, H, W = x.shape

    def _cidx(w, j):
        return jnp.where(w == 0, 56, jnp.where(w == 1, 35, 12)) + j

    grid = (len(_WIN_BASES), B, H // _HBLK, _NJ)

    out = pl.pallas_call(
        _body,
        grid=grid,
        in_specs=[
            pl.BlockSpec(
                (1, _CBLK, _HBLK, W),
                lambda w, b, h, j: (b, _cidx(w, j), h, 0),
            )
        ],
        out_specs=pl.BlockSpec(
            (1, 1, _HBLK, W),
            lambda w, b, h, j: (b, w, h, 0),
        ),
        out_shape=jax.ShapeDtypeStruct((B, len(_WIN_BASES), H, W), x.dtype),
    )(x)
    return out
